# Initial kernel scaffold; baseline (speedup 1.0000x reference)
import jax
import jax.numpy as jnp
from jax.experimental import pallas as pl


def kernel(clip_embeddings_list, W_imp, b_imp):
    raise NotImplementedError("write your pallas kernel here")


# speedup vs baseline: 8.7197x; 8.7197x over previous
import functools

import jax
import jax.numpy as jnp
from jax import lax
from jax.experimental import pallas as pl


N_TOK = 1024
N_KEEP = 512
D = 96


def _select_body(tok_ref, w_ref, out_ref):
    tokens = tok_ref[0, 0]
    w = w_ref[...]
    tok_r = tokens.astype(jnp.bfloat16).astype(jnp.float32)
    w_r = w.astype(jnp.bfloat16).astype(jnp.float32)
    s_col = lax.dot_general(tok_r, w_r, (((1,), (1,)), ((), ())),
                            preferred_element_type=jnp.float32)
    i_col = lax.broadcasted_iota(jnp.int32, (N_TOK, 1), 0)
    i_row = lax.broadcasted_iota(jnp.int32, (1, N_TOK), 1)
    eye = (i_col == i_row).astype(jnp.float32)
    s_row = lax.dot_general(s_col, eye, (((0,), (0,)), ((), ())),
                            preferred_element_type=jnp.float32,
                            precision=lax.Precision.HIGHEST)
    beats = (s_col > s_row) | ((s_col == s_row) & (i_col < i_row))
    rank_row = jnp.sum(beats.astype(jnp.float32), axis=0, keepdims=True)
    r_col = lax.broadcasted_iota(jnp.int32, (N_KEEP, 1), 0).astype(jnp.float32)
    sel = (rank_row == r_col).astype(jnp.float32)
    out_ref[0, 0] = lax.dot_general(sel, tokens, (((1,), (0,)), ((), ())),
                                    preferred_element_type=jnp.float32,
                                    precision=lax.Precision.HIGHEST)


def _first_argmax_col(m):
    mx = jnp.max(m, axis=1, keepdims=True)
    j = lax.broadcasted_iota(jnp.int32, m.shape, 1)
    return jnp.min(jnp.where(m == mx, j, m.shape[1]), axis=1, keepdims=True)


def _normalize(x):
    n = jnp.sqrt(jnp.sum(x * x, axis=1, keepdims=True))
    return x / jnp.maximum(n, 1e-8)


def _pairs_body(p1_ref, p2_ref, out_ref):
    p1 = p1_ref[0, 0]
    p2 = p2_ref[0, 0]
    sim = lax.dot_general(_normalize(p1), _normalize(p2),
                          (((1,), (1,)), ((), ())),
                          preferred_element_type=jnp.float32)
    top = _first_argmax_col(sim)
    j_row = lax.broadcasted_iota(jnp.int32, (N_KEEP, N_KEEP), 1)
    g1 = (top == j_row).astype(jnp.float32)
    merged = lax.dot_general(g1, (p1 + p2) * 0.5, (((1,), (0,)), ((), ())),
                             preferred_element_type=jnp.float32,
                             precision=lax.Precision.HIGHEST)
    sn = _normalize(merged)
    sim2 = lax.dot_general(sn, sn, (((1,), (1,)), ((), ())),
                           preferred_element_type=jnp.float32)
    top2 = _first_argmax_col(sim2)
    counts = jnp.sum((top2 == j_row).astype(jnp.float32), axis=0,
                     keepdims=True)
    out_ref[0, 0] = lax.dot_general(counts, merged, (((1,), (0,)), ((), ())),
                                    preferred_element_type=jnp.float32,
                                    precision=lax.Precision.HIGHEST) * (1.0 / N_KEEP)


@jax.jit
def kernel(clip_embeddings_list, W_imp, b_imp):
    del b_imp
    n_clips, batch = clip_embeddings_list.shape[:2]

    processed = pl.pallas_call(
        _select_body,
        grid=(n_clips, batch),
        in_specs=[
            pl.BlockSpec((1, 1, N_TOK, D), lambda c, b: (c, b, 0, 0)),
            pl.BlockSpec((1, D), lambda c, b: (0, 0)),
        ],
        out_specs=pl.BlockSpec((1, 1, N_KEEP, D), lambda c, b: (c, b, 0, 0)),
        out_shape=jax.ShapeDtypeStruct((n_clips, batch, N_KEEP, D), jnp.float32),
    )(clip_embeddings_list, W_imp)

    out = pl.pallas_call(
        _pairs_body,
        grid=(n_clips - 1, batch),
        in_specs=[
            pl.BlockSpec((1, 1, N_KEEP, D), lambda p, b: (p, b, 0, 0)),
            pl.BlockSpec((1, 1, N_KEEP, D), lambda p, b: (p + 1, b, 0, 0)),
        ],
        out_specs=pl.BlockSpec((1, 1, 1, D), lambda p, b: (p, b, 0, 0)),
        out_shape=jax.ShapeDtypeStruct((n_clips - 1, batch, 1, D), jnp.float32),
    )(processed, processed)
    return out
